# speedup vs baseline: 4.4481x; 4.4481x over previous
import functools

import jax
import jax.numpy as jnp
from jax import lax
from jax.experimental import pallas as pl
from jax.experimental.pallas import tpu as pltpu
from jax.experimental.pallas import tpu_sc as plsc

N = 320000
D = 128
NSYS = 10000
NC = 2
NS = 16
CHUNK = 128
NCHUNKS = N // CHUNK
CHUNKS_PER_SC = NCHUNKS // NC
WB = 40
NWB = NSYS // WB


def _sc_partial_sums(features, batch_index):
    mesh = plsc.VectorSubcoreMesh(core_axis_name="c", subcore_axis_name="s")

    @functools.partial(
        pl.kernel,
        out_type=jax.ShapeDtypeStruct((NC, NSYS, D), jnp.float32),
        mesh=mesh,
        scratch_types=[
            pltpu.VMEM((CHUNK, D), jnp.float32),
            pltpu.VMEM((CHUNK,), jnp.int32),
            pltpu.VMEM((WB, D), jnp.float32),
            pltpu.VMEM_SHARED((NSYS, D), jnp.float32),
        ],
    )
    def body(feat_hbm, idx_hbm, out_hbm, row_v, idx_v, zero_v, acc):
        c = lax.axis_index("c")
        t = lax.axis_index("s")

        def zrow(i, _):
            for k in range(D // 16):
                zero_v[i, pl.ds(16 * k, 16)] = jnp.zeros((16,), jnp.float32)
            return 0

        lax.fori_loop(0, WB, zrow, 0)

        def zchunk(i, _):
            j = t + NS * i

            @pl.when(j < NWB)
            def _():
                pltpu.sync_copy(zero_v, acc.at[pl.ds(WB * j, WB)])

            return 0

        lax.fori_loop(0, (NWB + NS - 1) // NS, zchunk, 0)
        plsc.subcore_barrier()

        base = CHUNKS_PER_SC * c
        end = base + CHUNKS_PER_SC

        def achunk(i, _):
            j = base + t + NS * i

            @pl.when(j < end)
            def _():
                r = CHUNK * j
                pltpu.sync_copy(idx_hbm.at[pl.ds(r, CHUNK)], idx_v)
                pltpu.sync_copy(feat_hbm.at[pl.ds(r, CHUNK)], row_v)
                pltpu.sync_copy(row_v, acc.at[idx_v], add=True)

            return 0

        lax.fori_loop(0, (CHUNKS_PER_SC + NS - 1) // NS, achunk, 0)
        plsc.subcore_barrier()

        def wchunk(i, _):
            j = t + NS * i

            @pl.when(j < NWB)
            def _():
                pltpu.sync_copy(
                    acc.at[pl.ds(WB * j, WB)], out_hbm.at[c, pl.ds(WB * j, WB)]
                )

            return 0

        lax.fori_loop(0, (NWB + NS - 1) // NS, wchunk, 0)

    return body(features, batch_index)


def _tc_combine(partials):
    BS = 1000

    def body(p_ref, o_ref):
        o_ref[...] = p_ref[0] + p_ref[1]

    return pl.pallas_call(
        body,
        out_shape=jax.ShapeDtypeStruct((NSYS, D), jnp.float32),
        grid=(NSYS // BS,),
        in_specs=[pl.BlockSpec((NC, BS, D), lambda i: (0, i, 0))],
        out_specs=pl.BlockSpec((BS, D), lambda i: (i, 0)),
    )(partials)


def kernel(features, batch_index, natoms):
    del natoms
    bi = batch_index.astype(jnp.int32)
    partials = _sc_partial_sums(features, bi)
    return _tc_combine(partials)


# speedup vs baseline: 7.5030x; 1.6868x over previous
import functools

import jax
import jax.numpy as jnp
from jax import lax
from jax.experimental import pallas as pl
from jax.experimental.pallas import tpu as pltpu
from jax.experimental.pallas import tpu_sc as plsc

N = 320000
D = 128
NSYS = 10000
NC = 2
NS = 16
CHUNK = 128
NCHUNKS = N // CHUNK
CHUNKS_PER_SC = NCHUNKS // NC
PER_TILE = (CHUNKS_PER_SC // NS) & ~1
WB = 40
NWB = NSYS // WB


def _sc_partial_sums(features, batch_index):
    mesh = plsc.VectorSubcoreMesh(core_axis_name="c", subcore_axis_name="s")

    @functools.partial(
        pl.kernel,
        out_type=jax.ShapeDtypeStruct((NC, NSYS, D), jnp.float32),
        mesh=mesh,
        scratch_types=[
            pltpu.VMEM((2, CHUNK, D), jnp.float32),
            pltpu.VMEM((2, CHUNK), jnp.int32),
            pltpu.VMEM((WB, D), jnp.float32),
            pltpu.VMEM_SHARED((NSYS, D), jnp.float32),
            pltpu.SemaphoreType.DMA,
            pltpu.SemaphoreType.DMA,
        ],
    )
    def body(feat_hbm, idx_hbm, out_hbm, row_v, idx_v, zero_v, acc, sem0, sem1):
        c = lax.axis_index("c")
        t = lax.axis_index("s")

        def zrow(i, _):
            for k in range(D // 16):
                zero_v[i, pl.ds(16 * k, 16)] = jnp.zeros((16,), jnp.float32)
            return 0

        lax.fori_loop(0, WB, zrow, 0)

        def zchunk(i, _):
            j = t + NS * i

            @pl.when(j < NWB)
            def _():
                pltpu.sync_copy(zero_v, acc.at[pl.ds(WB * j, WB)])

            return 0

        lax.fori_loop(0, (NWB + NS - 1) // NS, zchunk, 0)
        plsc.subcore_barrier()

        base = CHUNKS_PER_SC * c
        sems = (sem0, sem1)

        def start_gather(i, p):
            r = CHUNK * (base + t + NS * i)
            pltpu.async_copy(idx_hbm.at[pl.ds(r, CHUNK)], idx_v.at[p], sems[p])
            pltpu.async_copy(feat_hbm.at[pl.ds(r, CHUNK)], row_v.at[p], sems[p])

        def wait_gather(i, p):
            r = CHUNK * (base + t + NS * i)
            pltpu.make_async_copy(
                idx_hbm.at[pl.ds(r, CHUNK)], idx_v.at[p], sems[p]
            ).wait()
            pltpu.make_async_copy(
                feat_hbm.at[pl.ds(r, CHUNK)], row_v.at[p], sems[p]
            ).wait()

        start_gather(0, 0)
        start_gather(1, 1)

        def achunk(i2, _):
            for p in range(2):
                i = 2 * i2 + p
                wait_gather(i, p)
                pltpu.sync_copy(row_v.at[p], acc.at[idx_v.at[p]], add=True)

                @pl.when(i + 2 < PER_TILE)
                def _():
                    start_gather(i + 2, p)

            return 0

        lax.fori_loop(0, PER_TILE // 2, achunk, 0)

        @pl.when(t < CHUNKS_PER_SC - NS * PER_TILE)
        def _():
            r = CHUNK * (base + NS * PER_TILE + t)
            pltpu.sync_copy(idx_hbm.at[pl.ds(r, CHUNK)], idx_v.at[0])
            pltpu.sync_copy(feat_hbm.at[pl.ds(r, CHUNK)], row_v.at[0])
            pltpu.sync_copy(row_v.at[0], acc.at[idx_v.at[0]], add=True)

        plsc.subcore_barrier()

        def wchunk(i, _):
            j = t + NS * i

            @pl.when(j < NWB)
            def _():
                pltpu.sync_copy(
                    acc.at[pl.ds(WB * j, WB)], out_hbm.at[c, pl.ds(WB * j, WB)]
                )

            return 0

        lax.fori_loop(0, (NWB + NS - 1) // NS, wchunk, 0)

    return body(features, batch_index)


def _tc_combine(partials):
    BS = 1000

    def body(p_ref, o_ref):
        o_ref[...] = p_ref[0] + p_ref[1]

    return pl.pallas_call(
        body,
        out_shape=jax.ShapeDtypeStruct((NSYS, D), jnp.float32),
        grid=(NSYS // BS,),
        in_specs=[pl.BlockSpec((NC, BS, D), lambda i: (0, i, 0))],
        out_specs=pl.BlockSpec((BS, D), lambda i: (i, 0)),
    )(partials)


def kernel(features, batch_index, natoms):
    del natoms
    bi = batch_index.astype(jnp.int32)
    partials = _sc_partial_sums(features, bi)
    return _tc_combine(partials)
